# Initial kernel scaffold; baseline (speedup 1.0000x reference)
import jax
import jax.numpy as jnp
from jax.experimental import pallas as pl


def kernel(speed_seq, cluster_id):
    raise NotImplementedError("write your pallas kernel here")


# speedup vs baseline: 2.1959x; 2.1959x over previous
import jax
import jax.numpy as jnp
from jax.experimental import pallas as pl
from jax.experimental.pallas import tpu as pltpu

B, T, N, F = 16, 20, 50000, 2
H = 10
R = 64
NB = 512
NSTEPS = (N + NB - 1) // NB
NPAD = NSTEPS * NB


def _fused_kernel(x_ref, cid_ref, out_pred_ref, out_reg_ref, acc_ref, cnt_ref):
    i = pl.program_id(0)

    s = jnp.sum(x_ref[...], axis=1) * (1.0 / T)
    pred = s.reshape(B, NB, F)[:, :, 0]

    node_idx = i * NB + jax.lax.broadcasted_iota(jnp.int32, (1, NB), 1)
    valid = node_idx < N
    pred = jnp.where(valid, pred, 0.0)

    out_pred_ref[...] = jnp.broadcast_to(pred[:, None, :], (B, H, NB))

    cid = cid_ref[0, 0, :]
    rows = jax.lax.broadcasted_iota(jnp.int32, (R, NB), 0)
    onehot_t = (rows == cid[None, :]).astype(jnp.float32)

    @pl.when(i == 0)
    def _init():
        acc_ref[...] = jnp.zeros_like(acc_ref)
        cnt_ref[...] = jnp.zeros_like(cnt_ref)

    seg = jax.lax.dot_general(
        pred, onehot_t, (((1,), (1,)), ((), ())),
        preferred_element_type=jnp.float32)
    acc_ref[...] += seg
    cnt_ref[...] += jnp.sum(onehot_t, axis=1)[None, :]

    @pl.when(i == NSTEPS - 1)
    def _finish():
        regional = acc_ref[...] / cnt_ref[...]
        out_reg_ref[...] = jnp.broadcast_to(regional[:, None, :], (B, H, R))


def kernel(speed_seq, cluster_id):
    x = speed_seq.reshape(B, T, F * N)
    cid = cluster_id.astype(jnp.int32)
    cid_pad = jnp.concatenate(
        [cid, jnp.full((NPAD - N,), R, dtype=jnp.int32)]).reshape(NSTEPS, 1, NB)

    pred_speed, regional = pl.pallas_call(
        _fused_kernel,
        grid=(NSTEPS,),
        in_specs=[
            pl.BlockSpec((B, T, F * NB), lambda i: (0, 0, i)),
            pl.BlockSpec((1, 1, NB), lambda i: (i, 0, 0)),
        ],
        out_specs=[
            pl.BlockSpec((B, H, NB), lambda i: (0, 0, i)),
            pl.BlockSpec((B, H, R), lambda i: (0, 0, 0)),
        ],
        out_shape=[
            jax.ShapeDtypeStruct((B, H, N), jnp.float32),
            jax.ShapeDtypeStruct((B, H, R), jnp.float32),
        ],
        scratch_shapes=[
            pltpu.VMEM((B, R), jnp.float32),
            pltpu.VMEM((1, R), jnp.float32),
        ],
    )(x, cid_pad)
    return pred_speed, regional


# speedup vs baseline: 3.1440x; 1.4318x over previous
import jax
import jax.numpy as jnp
from jax.experimental import pallas as pl
from jax.experimental.pallas import tpu as pltpu

B, T, N, F = 16, 20, 50000, 2
H = 10
R = 64
NB = 512
NSTEPS = (N + NB - 1) // NB
NPAD = NSTEPS * NB


def _fused_kernel(x_ref, cid_ref, out_pred_ref, out_reg_ref, acc_ref, cnt_ref,
                  d_ref):
    i = pl.program_id(0)

    @pl.when(i == 0)
    def _build_d():
        rows = jax.lax.broadcasted_iota(jnp.int32, (F * NB, NB), 0)
        cols = jax.lax.broadcasted_iota(jnp.int32, (F * NB, NB), 1)
        d_ref[...] = (rows == F * cols).astype(jnp.float32)

    wr = jax.lax.broadcasted_iota(jnp.int32, (B, B * T), 0)
    wc = jax.lax.broadcasted_iota(jnp.int32, (B, B * T), 1)
    w = jnp.where(wc // T == wr, 1.0 / T, 0.0)
    pred2 = jax.lax.dot_general(
        w, x_ref[...], (((1,), (0,)), ((), ())),
        preferred_element_type=jnp.float32)
    col_idx = i * (F * NB) + jax.lax.broadcasted_iota(jnp.int32, (1, F * NB), 1)
    pred2 = jnp.where(col_idx < F * N, pred2, 0.0)
    pred = jax.lax.dot_general(
        pred2, d_ref[...], (((1,), (0,)), ((), ())),
        preferred_element_type=jnp.float32)

    node_idx = i * NB + jax.lax.broadcasted_iota(jnp.int32, (1, NB), 1)
    valid = node_idx < N
    pred = jnp.where(valid, pred, 0.0)

    out_pred_ref[...] = jnp.broadcast_to(pred[:, None, :], (B, H, NB))

    cid = cid_ref[0, 0, :]
    rows = jax.lax.broadcasted_iota(jnp.int32, (R, NB), 0)
    onehot_t = (rows == cid[None, :]).astype(jnp.float32)

    @pl.when(i == 0)
    def _init():
        acc_ref[...] = jnp.zeros_like(acc_ref)
        cnt_ref[...] = jnp.zeros_like(cnt_ref)

    seg = jax.lax.dot_general(
        pred, onehot_t, (((1,), (1,)), ((), ())),
        preferred_element_type=jnp.float32)
    acc_ref[...] += seg
    cnt_ref[...] += jnp.sum(onehot_t, axis=1)[None, :]

    @pl.when(i == NSTEPS - 1)
    def _finish():
        regional = acc_ref[...] / cnt_ref[...]
        out_reg_ref[...] = jnp.broadcast_to(regional[:, None, :], (B, H, R))


def kernel(speed_seq, cluster_id):
    x = speed_seq.reshape(B * T, F * N)
    cid = cluster_id.astype(jnp.int32)
    cid_pad = jnp.concatenate(
        [cid, jnp.full((NPAD - N,), R, dtype=jnp.int32)]).reshape(NSTEPS, 1, NB)

    pred_speed, regional = pl.pallas_call(
        _fused_kernel,
        grid=(NSTEPS,),
        in_specs=[
            pl.BlockSpec((B * T, F * NB), lambda i: (0, i)),
            pl.BlockSpec((1, 1, NB), lambda i: (i, 0, 0)),
        ],
        out_specs=[
            pl.BlockSpec((B, H, NB), lambda i: (0, 0, i)),
            pl.BlockSpec((B, H, R), lambda i: (0, 0, 0)),
        ],
        out_shape=[
            jax.ShapeDtypeStruct((B, H, N), jnp.float32),
            jax.ShapeDtypeStruct((B, H, R), jnp.float32),
        ],
        scratch_shapes=[
            pltpu.VMEM((B, R), jnp.float32),
            pltpu.VMEM((1, R), jnp.float32),
            pltpu.VMEM((F * NB, NB), jnp.float32),
        ],
    )(x, cid_pad)
    return pred_speed, regional


# speedup vs baseline: 13.3446x; 4.2445x over previous
import jax
import jax.numpy as jnp
from jax.experimental import pallas as pl
from jax.experimental.pallas import tpu as pltpu

B, T, N, F = 16, 20, 50000, 2
H = 10
R = 64
NB = 512
NSTEPS = (N + NB - 1) // NB
NPAD = NSTEPS * NB


def _fused_kernel(x_ref, cid_ref, out_pred_ref, out_reg_ref, acc_ref, cnt_ref):
    i = pl.program_id(0)

    wr = jax.lax.broadcasted_iota(jnp.int32, (B, B * T), 0)
    wc = jax.lax.broadcasted_iota(jnp.int32, (B, B * T), 1)
    w = jnp.where(wc // T == wr, 1.0 / T, 0.0)
    pred = jax.lax.dot_general(
        w, x_ref[...], (((1,), (0,)), ((), ())),
        preferred_element_type=jnp.float32)

    node_idx = i * NB + jax.lax.broadcasted_iota(jnp.int32, (1, NB), 1)
    valid = node_idx < N
    pred = jnp.where(valid, pred, 0.0)

    out_pred_ref[...] = jnp.broadcast_to(pred[None, :, :], (H, B, NB))

    cid = cid_ref[0, 0, :]
    rows = jax.lax.broadcasted_iota(jnp.int32, (R, NB), 0)
    onehot_t = (rows == cid[None, :]).astype(jnp.float32)

    @pl.when(i == 0)
    def _init():
        acc_ref[...] = jnp.zeros_like(acc_ref)
        cnt_ref[...] = jnp.zeros_like(cnt_ref)

    seg = jax.lax.dot_general(
        pred, onehot_t, (((1,), (1,)), ((), ())),
        preferred_element_type=jnp.float32)
    acc_ref[...] += seg
    cnt_ref[...] += jnp.sum(onehot_t, axis=1)[None, :]

    @pl.when(i == NSTEPS - 1)
    def _finish():
        regional = acc_ref[...] / cnt_ref[...]
        out_reg_ref[...] = jnp.broadcast_to(regional[None, :, :], (H, B, R))


def kernel(speed_seq, cluster_id):
    x0 = speed_seq.transpose(0, 1, 3, 2)[:, :, 0, :].reshape(B * T, N)
    cid = cluster_id.astype(jnp.int32)
    cid_pad = jnp.concatenate(
        [cid, jnp.full((NPAD - N,), R, dtype=jnp.int32)]).reshape(NSTEPS, 1, NB)

    pred_speed, regional = pl.pallas_call(
        _fused_kernel,
        grid=(NSTEPS,),
        in_specs=[
            pl.BlockSpec((B * T, NB), lambda i: (0, i)),
            pl.BlockSpec((1, 1, NB), lambda i: (i, 0, 0)),
        ],
        out_specs=[
            pl.BlockSpec((H, B, NB), lambda i: (0, 0, i)),
            pl.BlockSpec((H, B, R), lambda i: (0, 0, 0)),
        ],
        out_shape=[
            jax.ShapeDtypeStruct((H, B, N), jnp.float32),
            jax.ShapeDtypeStruct((H, B, R), jnp.float32),
        ],
        scratch_shapes=[
            pltpu.VMEM((B, R), jnp.float32),
            pltpu.VMEM((1, R), jnp.float32),
        ],
    )(x0, cid_pad)
    return pred_speed.transpose(1, 0, 2), regional.transpose(1, 0, 2)


# speedup vs baseline: 19.2237x; 1.4406x over previous
import jax
import jax.numpy as jnp
from jax.experimental import pallas as pl
from jax.experimental.pallas import tpu as pltpu

B, T, N, F = 16, 20, 50000, 2
TP = 24
H = 10
R = 64
NB = 512
NSTEPS = (N + NB - 1) // NB
NPAD = NSTEPS * NB


def _fused_kernel(x_ref, cid_ref, out_pred_ref, out_reg_ref, acc_ref, cnt_ref):
    i = pl.program_id(0)

    wr = jax.lax.broadcasted_iota(jnp.int32, (B, B * T), 0)
    wc = jax.lax.broadcasted_iota(jnp.int32, (B, B * T), 1)
    w = jnp.where(wc // T == wr, 1.0 / T, 0.0)
    x = x_ref[...].reshape(B * T, NB)
    pred = jax.lax.dot_general(
        w, x, (((1,), (0,)), ((), ())),
        preferred_element_type=jnp.float32)

    node_idx = i * NB + jax.lax.broadcasted_iota(jnp.int32, (1, NB), 1)
    valid = node_idx < N
    pred = jnp.where(valid, pred, 0.0)

    out_pred_ref[...] = jnp.broadcast_to(pred[None, :, :], (H, B, NB))

    cid = cid_ref[0, 0, :]
    rows = jax.lax.broadcasted_iota(jnp.int32, (R, NB), 0)
    onehot_t = (rows == cid[None, :]).astype(jnp.float32)

    @pl.when(i == 0)
    def _init():
        acc_ref[...] = jnp.zeros_like(acc_ref)
        cnt_ref[...] = jnp.zeros_like(cnt_ref)

    seg = jax.lax.dot_general(
        pred, onehot_t, (((1,), (1,)), ((), ())),
        preferred_element_type=jnp.float32)
    acc_ref[...] += seg
    cnt_ref[...] += jnp.sum(onehot_t, axis=1)[None, :]

    @pl.when(i == NSTEPS - 1)
    def _finish():
        regional = acc_ref[...] / cnt_ref[...]
        out_reg_ref[...] = jnp.broadcast_to(regional[None, :, :], (H, B, R))


def kernel(speed_seq, cluster_id):
    x0 = speed_seq[:, :, :, 0]
    cid = cluster_id.astype(jnp.int32)
    cid_pad = jnp.concatenate(
        [cid, jnp.full((NPAD - N,), R, dtype=jnp.int32)]).reshape(NSTEPS, 1, NB)

    pred_speed, regional = pl.pallas_call(
        _fused_kernel,
        grid=(NSTEPS,),
        in_specs=[
            pl.BlockSpec((B, T, NB), lambda i: (0, 0, i)),
            pl.BlockSpec((1, 1, NB), lambda i: (i, 0, 0)),
        ],
        out_specs=[
            pl.BlockSpec((H, B, NB), lambda i: (0, 0, i)),
            pl.BlockSpec((H, B, R), lambda i: (0, 0, 0)),
        ],
        out_shape=[
            jax.ShapeDtypeStruct((H, B, N), jnp.float32),
            jax.ShapeDtypeStruct((H, B, R), jnp.float32),
        ],
        scratch_shapes=[
            pltpu.VMEM((B, R), jnp.float32),
            pltpu.VMEM((1, R), jnp.float32),
        ],
    )(x0, cid_pad)
    return pred_speed.transpose(1, 0, 2), regional.transpose(1, 0, 2)


# speedup vs baseline: 22.1989x; 1.1548x over previous
import jax
import jax.numpy as jnp
from jax.experimental import pallas as pl
from jax.experimental.pallas import tpu as pltpu

B, T, N, F = 16, 20, 50000, 2
TP = 24
H = 10
R = 64
NB = 1024
NSTEPS = (N + NB - 1) // NB
NPAD = NSTEPS * NB


def _fused_kernel(x_ref, cid_ref, out_pred_ref, out_reg_ref, acc_ref, cnt_ref):
    i = pl.program_id(0)

    wr = jax.lax.broadcasted_iota(jnp.int32, (B, B * T), 0)
    wc = jax.lax.broadcasted_iota(jnp.int32, (B, B * T), 1)
    w = jnp.where(wc // T == wr, 1.0 / T, 0.0)
    x = x_ref[...].reshape(B * T, NB)
    pred = jax.lax.dot_general(
        w, x, (((1,), (0,)), ((), ())),
        preferred_element_type=jnp.float32)

    node_idx = i * NB + jax.lax.broadcasted_iota(jnp.int32, (1, NB), 1)
    valid = node_idx < N
    pred = jnp.where(valid, pred, 0.0)

    out_pred_ref[...] = jnp.broadcast_to(pred[None, :, :], (H, B, NB))

    cid = cid_ref[0, 0, :]
    rows = jax.lax.broadcasted_iota(jnp.int32, (R, NB), 0)
    onehot_t = (rows == cid[None, :]).astype(jnp.float32)

    @pl.when(i == 0)
    def _init():
        acc_ref[...] = jnp.zeros_like(acc_ref)
        cnt_ref[...] = jnp.zeros_like(cnt_ref)

    seg = jax.lax.dot_general(
        pred, onehot_t, (((1,), (1,)), ((), ())),
        preferred_element_type=jnp.float32)
    acc_ref[...] += seg
    cnt_ref[...] += jnp.sum(onehot_t, axis=1)[None, :]

    @pl.when(i == NSTEPS - 1)
    def _finish():
        regional = acc_ref[...] / cnt_ref[...]
        out_reg_ref[...] = jnp.broadcast_to(regional[None, :, :], (H, B, R))


def kernel(speed_seq, cluster_id):
    x0 = speed_seq[:, :, :, 0]
    cid = cluster_id.astype(jnp.int32)
    cid_pad = jnp.concatenate(
        [cid, jnp.full((NPAD - N,), R, dtype=jnp.int32)]).reshape(NSTEPS, 1, NB)

    pred_speed, regional = pl.pallas_call(
        _fused_kernel,
        grid=(NSTEPS,),
        in_specs=[
            pl.BlockSpec((B, T, NB), lambda i: (0, 0, i)),
            pl.BlockSpec((1, 1, NB), lambda i: (i, 0, 0)),
        ],
        out_specs=[
            pl.BlockSpec((H, B, NB), lambda i: (0, 0, i)),
            pl.BlockSpec((H, B, R), lambda i: (0, 0, 0)),
        ],
        out_shape=[
            jax.ShapeDtypeStruct((H, B, N), jnp.float32),
            jax.ShapeDtypeStruct((H, B, R), jnp.float32),
        ],
        scratch_shapes=[
            pltpu.VMEM((B, R), jnp.float32),
            pltpu.VMEM((1, R), jnp.float32),
        ],
    )(x0, cid_pad)
    return pred_speed.transpose(1, 0, 2), regional.transpose(1, 0, 2)
